# baseline (speedup 1.0000x reference)
import functools

import jax
import jax.numpy as jnp
from jax import lax
from jax.experimental import pallas as pl
from jax.experimental.pallas import tpu as pltpu
from jax.experimental.pallas import tpu_sc as plsc

NC = 2
NS = 16
NW = NC * NS
L = 16
NSLOT = 12


def _rsqrt_nr(x):
    xi = plsc.bitcast(x, jnp.int32)
    yi = jnp.int32(0x5F3759DF) - (xi >> 1)
    y = plsc.bitcast(yi, jnp.float32)
    for _ in range(3):
        y = y * (jnp.float32(1.5) - jnp.float32(0.5) * x * y * y)
    return y


def _make_sc_call(B, D):
    b_per_w = B // NW
    groups = b_per_w // L
    mesh = plsc.VectorSubcoreMesh(
        core_axis_name="c", subcore_axis_name="s", num_cores=NC, num_subcores=NS
    )

    @functools.partial(
        pl.kernel,
        out_type=jax.ShapeDtypeStruct((B,), jnp.float32),
        mesh=mesh,
        compiler_params=pltpu.CompilerParams(
            needs_layout_passes=False, use_tc_tiling_on_sc=True),
        scratch_types=[
            pltpu.VMEM((b_per_w,), jnp.int32),
            pltpu.VMEM((b_per_w,), jnp.int32),
            pltpu.VMEM((NSLOT, D, 128), jnp.float32),
            pltpu.VMEM((NSLOT, D, 128), jnp.float32),
            pltpu.VMEM((b_per_w,), jnp.float32),
        ] + [pltpu.SemaphoreType.DMA] * (2 * NSLOT),
    )
    def sc_call(uid_hbm, iid_hbm, ut_hbm, it_hbm, out_hbm,
                uidx_v, iidx_v, uslots_v, islots_v, res_v, *sems):
        usems = sems[:NSLOT]
        isems = sems[NSLOT:]
        wid = lax.axis_index("s") * NC + lax.axis_index("c")
        base = wid * b_per_w

        pltpu.sync_copy(uid_hbm.at[pl.ds(base, b_per_w)], uidx_v)
        pltpu.sync_copy(iid_hbm.at[pl.ds(base, b_per_w)], iidx_v)

        lanes = lax.iota(jnp.int32, L)
        zeros16i = jnp.zeros((L,), jnp.int32)

        BW = NSLOT // 3

        def group_body(g, _):
            uvec = uidx_v[pl.ds(g * L, L)]
            ivec = iidx_v[pl.ds(g * L, L)]
            accd = jnp.zeros((L,), jnp.float32)
            accu = jnp.zeros((L,), jnp.float32)
            acci = jnp.zeros((L,), jnp.float32)
            nburst = L // BW
            copies = {}

            def issue(b):
                sh = (b % 3) * BW
                cs = []
                for t in range(BW):
                    j = b * BW + t
                    s = sh + t
                    cs.append(pltpu.async_copy(
                        ut_hbm.at[:, pl.ds((uvec[j] >> 7) * 128, 128)],
                        uslots_v.at[s], usems[s]))
                    cs.append(pltpu.async_copy(
                        it_hbm.at[:, pl.ds((ivec[j] >> 7) * 128, 128)],
                        islots_v.at[s], isems[s]))
                copies[b] = cs

            issue(0)
            issue(1)
            for b in range(nburst):
                if b + 2 < nburst:
                    issue(b + 2)
                for c in copies.pop(b):
                    c.wait()
                sh = (b % 3) * BW
                for t in range(BW):
                    j = b * BW + t
                    s = sh + t
                    ucol = zeros16i + (uvec[j] & 127)
                    icol = zeros16i + (ivec[j] & 127)
                    u0 = plsc.load_gather(uslots_v.at[s], [lanes, ucol])
                    u1 = plsc.load_gather(uslots_v.at[s], [lanes + L, ucol])
                    v0 = plsc.load_gather(islots_v.at[s], [lanes, icol])
                    v1 = plsc.load_gather(islots_v.at[s], [lanes + L, icol])
                    dot_s = jnp.sum(u0 * v0 + u1 * v1, axis=0)
                    nu2_s = jnp.sum(u0 * u0 + u1 * u1, axis=0)
                    ni2_s = jnp.sum(v0 * v0 + v1 * v1, axis=0)
                    sel = lanes == j
                    accd = jnp.where(sel, dot_s, accd)
                    accu = jnp.where(sel, nu2_s, accu)
                    acci = jnp.where(sel, ni2_s, acci)
            rnu = _rsqrt_nr(jnp.maximum(accu, jnp.float32(1e-16)))
            rni = _rsqrt_nr(jnp.maximum(acci, jnp.float32(1e-16)))
            res_v[pl.ds(g * L, L)] = accd * rnu * rni
            return 0

        lax.fori_loop(0, groups, group_body, 0)
        pltpu.sync_copy(res_v, out_hbm.at[pl.ds(base, b_per_w)])

    return sc_call


def kernel(user_id, item_id, user_table, item_table):
    B = user_id.shape[0]
    D = user_table.shape[1]
    uid = user_id.astype(jnp.int32)
    iid = item_id.astype(jnp.int32)
    return _make_sc_call(B, D)(uid, iid, user_table.T, item_table.T)


# speedup vs baseline: 1.0504x; 1.0504x over previous
import functools

import jax
import jax.numpy as jnp
from jax import lax
from jax.experimental import pallas as pl
from jax.experimental.pallas import tpu as pltpu
from jax.experimental.pallas import tpu_sc as plsc

NC = 2
NS = 16
NW = NC * NS
L = 16
NSLOT = 12


def _rsqrt_nr(x):
    xi = plsc.bitcast(x, jnp.int32)
    yi = jnp.int32(0x5F3759DF) - (xi >> 1)
    y = plsc.bitcast(yi, jnp.float32)
    for _ in range(3):
        y = y * (jnp.float32(1.5) - jnp.float32(0.5) * x * y * y)
    return y


def _make_sc_call(B, D):
    b_per_w = B // NW
    groups = b_per_w // L
    mesh = plsc.VectorSubcoreMesh(
        core_axis_name="c", subcore_axis_name="s", num_cores=NC, num_subcores=NS
    )

    @functools.partial(
        pl.kernel,
        out_type=jax.ShapeDtypeStruct((B,), jnp.float32),
        mesh=mesh,
        compiler_params=pltpu.CompilerParams(
            needs_layout_passes=False, use_tc_tiling_on_sc=True),
        scratch_types=[
            pltpu.VMEM((b_per_w,), jnp.int32),
            pltpu.VMEM((b_per_w,), jnp.int32),
            pltpu.VMEM((NSLOT, D, 128), jnp.float32),
            pltpu.VMEM((NSLOT, D, 128), jnp.float32),
            pltpu.VMEM((b_per_w,), jnp.float32),
        ] + [pltpu.SemaphoreType.DMA] * (2 * NSLOT),
    )
    def sc_call(uid_hbm, iid_hbm, ut_hbm, it_hbm, out_hbm,
                uidx_v, iidx_v, uslots_v, islots_v, res_v, *sems):
        usems = sems[:NSLOT]
        isems = sems[NSLOT:]
        wid = lax.axis_index("s") * NC + lax.axis_index("c")
        base = wid * b_per_w

        pltpu.sync_copy(uid_hbm.at[pl.ds(base, b_per_w)], uidx_v)
        pltpu.sync_copy(iid_hbm.at[pl.ds(base, b_per_w)], iidx_v)

        lanes = lax.iota(jnp.int32, L)
        zeros16i = jnp.zeros((L,), jnp.int32)

        BW = NSLOT // 3
        CHUNKS = 4

        def group_body(g, _):
            uvecs = [uidx_v[pl.ds((g * CHUNKS + c) * L, L)] for c in range(CHUNKS)]
            ivecs = [iidx_v[pl.ds((g * CHUNKS + c) * L, L)] for c in range(CHUNKS)]
            nburst = CHUNKS * L // BW
            copies = {}

            def issue(b):
                sh = (b % 3) * BW
                c = (b * BW) // L
                cs = []
                for t in range(BW):
                    lj = (b * BW + t) % L
                    s = sh + t
                    cs.append(pltpu.async_copy(
                        ut_hbm.at[:, pl.ds((uvecs[c][lj] >> 7) * 128, 128)],
                        uslots_v.at[s], usems[s]))
                    cs.append(pltpu.async_copy(
                        it_hbm.at[:, pl.ds((ivecs[c][lj] >> 7) * 128, 128)],
                        islots_v.at[s], isems[s]))
                copies[b] = cs

            issue(0)
            issue(1)
            accd = jnp.zeros((L,), jnp.float32)
            accu = jnp.zeros((L,), jnp.float32)
            acci = jnp.zeros((L,), jnp.float32)
            for b in range(nburst):
                if b + 2 < nburst:
                    issue(b + 2)
                for cp in copies.pop(b):
                    cp.wait()
                sh = (b % 3) * BW
                c = (b * BW) // L
                for t in range(BW):
                    lj = (b * BW + t) % L
                    s = sh + t
                    ucol = zeros16i + (uvecs[c][lj] & 127)
                    icol = zeros16i + (ivecs[c][lj] & 127)
                    u0 = plsc.load_gather(uslots_v.at[s], [lanes, ucol])
                    u1 = plsc.load_gather(uslots_v.at[s], [lanes + L, ucol])
                    v0 = plsc.load_gather(islots_v.at[s], [lanes, icol])
                    v1 = plsc.load_gather(islots_v.at[s], [lanes + L, icol])
                    dot_s = jnp.sum(u0 * v0 + u1 * v1, axis=0)
                    nu2_s = jnp.sum(u0 * u0 + u1 * u1, axis=0)
                    ni2_s = jnp.sum(v0 * v0 + v1 * v1, axis=0)
                    sel = lanes == lj
                    accd = jnp.where(sel, dot_s, accd)
                    accu = jnp.where(sel, nu2_s, accu)
                    acci = jnp.where(sel, ni2_s, acci)
                if (b * BW + BW) % L == 0:
                    rnu = _rsqrt_nr(jnp.maximum(accu, jnp.float32(1e-16)))
                    rni = _rsqrt_nr(jnp.maximum(acci, jnp.float32(1e-16)))
                    res_v[pl.ds((g * CHUNKS + c) * L, L)] = accd * rnu * rni
                    accd = jnp.zeros((L,), jnp.float32)
                    accu = jnp.zeros((L,), jnp.float32)
                    acci = jnp.zeros((L,), jnp.float32)
            return 0

        lax.fori_loop(0, groups // CHUNKS, group_body, 0)
        pltpu.sync_copy(res_v, out_hbm.at[pl.ds(base, b_per_w)])

    return sc_call


def kernel(user_id, item_id, user_table, item_table):
    B = user_id.shape[0]
    D = user_table.shape[1]
    uid = user_id.astype(jnp.int32)
    iid = item_id.astype(jnp.int32)
    return _make_sc_call(B, D)(uid, iid, user_table.T, item_table.T)


# speedup vs baseline: 1.0513x; 1.0009x over previous
import functools

import jax
import jax.numpy as jnp
from jax import lax
from jax.experimental import pallas as pl
from jax.experimental.pallas import tpu as pltpu
from jax.experimental.pallas import tpu_sc as plsc

NC = 2
NS = 16
NW = NC * NS
L = 16
NSLOT = 12


def _rsqrt_nr(x):
    xi = plsc.bitcast(x, jnp.int32)
    yi = jnp.int32(0x5F3759DF) - (xi >> 1)
    y = plsc.bitcast(yi, jnp.float32)
    for _ in range(3):
        y = y * (jnp.float32(1.5) - jnp.float32(0.5) * x * y * y)
    return y


def _make_sc_call(B, D):
    b_per_w = B // NW
    groups = b_per_w // L
    mesh = plsc.VectorSubcoreMesh(
        core_axis_name="c", subcore_axis_name="s", num_cores=NC, num_subcores=NS
    )

    @functools.partial(
        pl.kernel,
        out_type=jax.ShapeDtypeStruct((B,), jnp.float32),
        mesh=mesh,
        compiler_params=pltpu.CompilerParams(
            needs_layout_passes=False, use_tc_tiling_on_sc=True),
        scratch_types=[
            pltpu.VMEM((b_per_w,), jnp.int32),
            pltpu.VMEM((b_per_w,), jnp.int32),
            pltpu.VMEM((NSLOT, D, 128), jnp.float32),
            pltpu.VMEM((NSLOT, D, 128), jnp.float32),
            pltpu.VMEM((b_per_w,), jnp.float32),
        ] + [pltpu.SemaphoreType.DMA] * (2 * NSLOT),
    )
    def sc_call(uid_hbm, iid_hbm, ut_hbm, it_hbm, out_hbm,
                uidx_v, iidx_v, uslots_v, islots_v, res_v, *sems):
        usems = sems[:NSLOT]
        isems = sems[NSLOT:]
        wid = lax.axis_index("s") * NC + lax.axis_index("c")
        base = wid * b_per_w

        pltpu.sync_copy(uid_hbm.at[pl.ds(base, b_per_w)], uidx_v)
        pltpu.sync_copy(iid_hbm.at[pl.ds(base, b_per_w)], iidx_v)

        lanes = lax.iota(jnp.int32, L)
        zeros16i = jnp.zeros((L,), jnp.int32)

        BW = NSLOT // 3
        CHUNKS = 8

        def group_body(g, _):
            uvecs = [uidx_v[pl.ds((g * CHUNKS + c) * L, L)] for c in range(CHUNKS)]
            ivecs = [iidx_v[pl.ds((g * CHUNKS + c) * L, L)] for c in range(CHUNKS)]
            nburst = CHUNKS * L // BW
            copies = {}

            def issue(b):
                sh = (b % 3) * BW
                c = (b * BW) // L
                cs = []
                for t in range(BW):
                    lj = (b * BW + t) % L
                    s = sh + t
                    cs.append(pltpu.async_copy(
                        ut_hbm.at[:, pl.ds((uvecs[c][lj] >> 7) * 128, 128)],
                        uslots_v.at[s], usems[s]))
                    cs.append(pltpu.async_copy(
                        it_hbm.at[:, pl.ds((ivecs[c][lj] >> 7) * 128, 128)],
                        islots_v.at[s], isems[s]))
                copies[b] = cs

            issue(0)
            issue(1)
            accd = jnp.zeros((L,), jnp.float32)
            accu = jnp.zeros((L,), jnp.float32)
            acci = jnp.zeros((L,), jnp.float32)
            for b in range(nburst):
                if b + 2 < nburst:
                    issue(b + 2)
                for cp in copies.pop(b):
                    cp.wait()
                sh = (b % 3) * BW
                c = (b * BW) // L
                for t in range(BW):
                    lj = (b * BW + t) % L
                    s = sh + t
                    ucol = zeros16i + (uvecs[c][lj] & 127)
                    icol = zeros16i + (ivecs[c][lj] & 127)
                    u0 = plsc.load_gather(uslots_v.at[s], [lanes, ucol])
                    u1 = plsc.load_gather(uslots_v.at[s], [lanes + L, ucol])
                    v0 = plsc.load_gather(islots_v.at[s], [lanes, icol])
                    v1 = plsc.load_gather(islots_v.at[s], [lanes + L, icol])
                    dot_s = jnp.sum(u0 * v0 + u1 * v1, axis=0)
                    nu2_s = jnp.sum(u0 * u0 + u1 * u1, axis=0)
                    ni2_s = jnp.sum(v0 * v0 + v1 * v1, axis=0)
                    sel = lanes == lj
                    accd = jnp.where(sel, dot_s, accd)
                    accu = jnp.where(sel, nu2_s, accu)
                    acci = jnp.where(sel, ni2_s, acci)
                if (b * BW + BW) % L == 0:
                    rnu = _rsqrt_nr(jnp.maximum(accu, jnp.float32(1e-16)))
                    rni = _rsqrt_nr(jnp.maximum(acci, jnp.float32(1e-16)))
                    res_v[pl.ds((g * CHUNKS + c) * L, L)] = accd * rnu * rni
                    accd = jnp.zeros((L,), jnp.float32)
                    accu = jnp.zeros((L,), jnp.float32)
                    acci = jnp.zeros((L,), jnp.float32)
            return 0

        lax.fori_loop(0, groups // CHUNKS, group_body, 0)
        pltpu.sync_copy(res_v, out_hbm.at[pl.ds(base, b_per_w)])

    return sc_call


def kernel(user_id, item_id, user_table, item_table):
    B = user_id.shape[0]
    D = user_table.shape[1]
    uid = user_id.astype(jnp.int32)
    iid = item_id.astype(jnp.int32)
    return _make_sc_call(B, D)(uid, iid, user_table.T, item_table.T)
